# baseline (speedup 1.0000x reference)
import jax
import jax.numpy as jnp
from jax import lax
from jax.experimental import pallas as pl
from jax.experimental.pallas import tpu as pltpu

_ROWS = 16384
_COLS = 8192
_NIDX = 128
_RB = 256


def _body(idx_ref, x_ref, o_ref, s_ref):
    @pl.when(pl.program_id(0) == 0)
    def _():
        col = lax.broadcasted_iota(jnp.int32, (_COLS, _NIDX), 0)
        s_ref[...] = (col == idx_ref[...]).astype(jnp.float32)

    o_ref[...] = lax.dot_general(
        x_ref[...],
        s_ref[...],
        (((1,), (0,)), ((), ())),
        preferred_element_type=jnp.float32,
        precision=lax.Precision.HIGHEST,
    )


def kernel(x, indices):
    return pl.pallas_call(
        _body,
        grid=(_ROWS // _RB,),
        in_specs=[
            pl.BlockSpec((1, _NIDX), lambda i: (0, 0)),
            pl.BlockSpec((_RB, _COLS), lambda i: (i, 0)),
        ],
        out_specs=pl.BlockSpec((_RB, _NIDX), lambda i: (i, 0)),
        out_shape=jax.ShapeDtypeStruct((_ROWS, _NIDX), jnp.float32),
        scratch_shapes=[pltpu.VMEM((_COLS, _NIDX), jnp.float32)],
        compiler_params=pltpu.CompilerParams(
            dimension_semantics=("arbitrary",),
        ),
    )(indices.reshape(1, _NIDX), x)
